# baseline (speedup 1.0000x reference)
import jax
import jax.numpy as jnp
from jax.experimental import pallas as pl
from jax.experimental.pallas import tpu as pltpu

PAD = 0.0


def _pad_mask_kernel(x_hbm, x_vmem, out_hbm, mask_vmem, sem_copy):
    data_copy = pltpu.make_async_copy(x_hbm, out_hbm, sem_copy)
    data_copy.start()
    mask_vmem[...] = x_vmem[...] == PAD
    data_copy.wait()


def kernel(tokens_batch):
    B, L = tokens_batch.shape
    out, mask = pl.pallas_call(
        _pad_mask_kernel,
        in_specs=[
            pl.BlockSpec(memory_space=pl.ANY),
            pl.BlockSpec((B, L), lambda: (0, 0)),
        ],
        out_specs=(
            pl.BlockSpec(memory_space=pl.ANY),
            pl.BlockSpec((B, L), lambda: (0, 0)),
        ),
        out_shape=(
            jax.ShapeDtypeStruct((B, L), tokens_batch.dtype),
            jax.ShapeDtypeStruct((B, L), jnp.bool_),
        ),
        scratch_shapes=[pltpu.SemaphoreType.DMA],
    )(tokens_batch, tokens_batch)
    return (out, mask)
